# baseline (speedup 1.0000x reference)
import functools

import jax
import jax.numpy as jnp
from jax.experimental import pallas as pl
from jax.experimental.pallas import tpu as pltpu

_CH = 12500
_NB = 4


def _body(cap, size, s_ref, mem_ref, val_ref, out_ref, bufs, isem, osem, vsem):
    nch = cap // _CH
    d_in = [
        pltpu.make_async_copy(
            mem_ref.at[pl.ds(c * _CH, _CH), :], bufs.at[c % _NB], isem.at[c % _NB]
        )
        for c in range(nch)
    ]
    d_out = [
        pltpu.make_async_copy(
            bufs.at[c % _NB], out_ref.at[pl.ds(c * _CH, _CH), :], osem.at[c % _NB]
        )
        for c in range(nch)
    ]
    for c in range(nch):
        if c >= _NB:
            d_out[c - _NB].wait()
        d_in[c].start()
        if c >= 1:
            d_in[c - 1].wait()
            d_out[c - 1].start()
    d_in[nch - 1].wait()
    d_out[nch - 1].start()
    for c in range(nch - _NB, nch):
        d_out[c].wait()

    s0 = s_ref[0]
    n1 = jnp.minimum(jnp.int32(size), cap - s0)
    nbits = size.bit_length()

    def segment(length, src_base, dst_base):
        copies = []
        off = jnp.int32(0)
        for k in reversed(range(nbits)):
            ln = 1 << k
            bit = (length & ln) != 0
            d = pltpu.make_async_copy(
                val_ref.at[pl.ds(src_base + off, ln), :],
                out_ref.at[pl.ds(dst_base + off, ln), :],
                vsem,
            )

            @pl.when(bit)
            def _start(d=d):
                d.start()

            copies.append((bit, d))
            off = off + jnp.where(bit, jnp.int32(ln), jnp.int32(0))
        return copies

    seg = segment(n1, jnp.int32(0), s0)
    seg += segment(jnp.int32(size) - n1, n1, jnp.int32(0))
    for bit, d in seg:

        @pl.when(bit)
        def _wait(d=d):
            d.wait()


def kernel(mem, val, store_index):
    cap, d = mem.shape
    size = min(val.shape[0], cap)
    assert cap % _CH == 0

    s0 = jnp.remainder(jnp.asarray(store_index, jnp.int32), cap).reshape(1)

    body = functools.partial(_body, cap, size)
    return pl.pallas_call(
        body,
        out_shape=jax.ShapeDtypeStruct((cap, d), mem.dtype),
        in_specs=[
            pl.BlockSpec(memory_space=pltpu.SMEM),
            pl.BlockSpec(memory_space=pl.ANY),
            pl.BlockSpec(memory_space=pl.ANY),
        ],
        out_specs=pl.BlockSpec(memory_space=pl.ANY),
        scratch_shapes=[
            pltpu.VMEM((_NB, _CH, d), jnp.float32),
            pltpu.SemaphoreType.DMA((_NB,)),
            pltpu.SemaphoreType.DMA((_NB,)),
            pltpu.SemaphoreType.DMA,
        ],
    )(s0, mem, val)
